# baseline (speedup 1.0000x reference)
import math
import jax
import jax.numpy as jnp
from jax.experimental import pallas as pl

HOP = 240
IR_LEN = 512
NFFT_PF = 768
SR = 24000.0
MAX_PULSES = 72000
IR_SCALE = 1.0
AP_SCALE = 0.005
PF_SCALE = 0.01


def _gen_noise(ap, key):
    b, hop, length = ap.shape
    exc = jax.random.uniform(key, (b, (length + 1) * hop), dtype=jnp.float32) - 0.5
    e2 = exc.reshape(b, length + 1, hop)
    frames = jnp.concatenate([e2[:, :-1], e2[:, 1:]], axis=2)
    spec = jnp.fft.rfft(frames, axis=2)
    spec = spec.at[:, :, 0].set(0.0)
    spec = spec.at[:, :, 1:].multiply(jnp.transpose(ap, (0, 2, 1)))
    nz = jnp.fft.irfft(spec, n=2 * hop, axis=2)
    n = jnp.arange(2 * hop, dtype=jnp.float32)
    hann = 0.5 * (1.0 - jnp.cos(2.0 * jnp.pi * n / (2 * hop)))
    nz = nz * hann
    out = jnp.zeros((b, length + 1, hop), dtype=jnp.float32)
    out = out.at[:, :length].add(nz[:, :, :hop]).at[:, 1:].add(nz[:, :, hop:])
    out = out.reshape(b, (length + 1) * hop)[:, : length * hop]
    return out, exc[:, : length * hop]


def _ola(ir_amp, ir_phase, window, pitch_samples, key):
    b, nfreq, length = ir_amp.shape
    ir_len = (nfreq - 1) * 2
    t = pitch_samples.shape[1]
    nf = pitch_samples / SR
    nf = nf.at[:, 0].set(jax.random.uniform(key, (b,), dtype=jnp.float32))
    phase = jnp.cumsum(nf, axis=1) % 1.0
    wrap = phase[:, :-1] > phase[:, 1:]
    flat = jnp.nonzero(wrap.ravel(), size=MAX_PULSES, fill_value=0)[0]
    valid = jnp.arange(MAX_PULSES) < jnp.sum(wrap)
    i0 = flat // (t - 1)
    i1 = flat % (t - 1)
    numer = 1.0 - phase[i0, i1]
    denom = numer + phase[i0, i1 + 1]
    frac = numer / jnp.where(denom > 0, denom, 1.0)
    frame = i1 // HOP
    amp = ir_amp[i0, :, frame]
    ph = ir_phase[i0, :, frame]
    dphase = jnp.arange(nfreq, dtype=jnp.float32)[None, :] * (-2.0 * jnp.pi / ir_len) * frac[:, None]
    spec = amp * jnp.exp(1j * (ph + dphase))
    pulses = jnp.fft.irfft(spec, n=ir_len, axis=1) * window
    pulses = pulses * valid[:, None].astype(pulses.dtype)
    out = jnp.zeros((b, t + ir_len), dtype=jnp.float32)
    out = out.at[i0[:, None], i1[:, None] + jnp.arange(ir_len)].add(pulses)
    return out[:, :t]


def _combine_kernel(p_ref, a_ref, y_ref):
    y_ref[...] = p_ref[...] + a_ref[...]


def kernel(ir, aperiodicity, post_filter, ir_window, pitch):
    b, c, length = ir.shape
    k1, k2 = jax.random.split(jax.random.key(42))
    ir = ir * IR_SCALE
    ir_amp = jnp.exp(ir[:, : c // 2 + 1])
    ir_phase = jnp.pad(ir[:, c // 2 + 1 :], ((0, 0), (1, 1), (0, 0)))
    ir_phase = ir_phase.at[:, 1::2].add(jnp.pi)
    pitch_s = jnp.repeat(pitch.astype(jnp.float32), HOP, axis=1)
    periodic = _ola(ir_amp, ir_phase, ir_window, pitch_s, k1)

    ap = aperiodicity * AP_SCALE
    aperiodic, noise_exc = _gen_noise(ap, k2)

    pf = post_filter * PF_SCALE
    pf = pf.at[:, 0, :].add(1.0)
    pf = jnp.transpose(pf, (0, 2, 1))
    PF = jnp.fft.rfft(pf, n=NFFT_PF, axis=2)

    def filt(sig):
        fr = sig.reshape(b, length, HOP)
        y = jnp.fft.irfft(jnp.fft.rfft(fr, n=NFFT_PF, axis=2) * PF, n=NFFT_PF, axis=2)
        m = (length - 1) * HOP + NFFT_PF
        idx = jnp.arange(length)[:, None] * HOP + jnp.arange(NFFT_PF)[None, :]
        out = jnp.zeros((b, m), dtype=jnp.float32).at[:, idx].add(y)
        return out[:, 120 : 120 + length * HOP]

    periodic = filt(periodic)
    aperiodic = filt(aperiodic)
    noise_exc = noise_exc[:, 120:]

    y2 = pl.pallas_call(
        _combine_kernel,
        out_shape=jax.ShapeDtypeStruct(periodic.shape, periodic.dtype),
    )(periodic, aperiodic)
    y = y2[:, None, :]
    return y, periodic, aperiodic, noise_exc


# speedup vs baseline: 1.1490x; 1.1490x over previous
import math

import jax
import jax.numpy as jnp
import numpy as np
from jax.experimental import pallas as pl
from jax.experimental.pallas import tpu as pltpu

HOP = 240
IR_LEN = 512
NFFT_PF = 768
SR = 24000.0
MAX_PULSES = 72000
IR_SCALE = 1.0
AP_SCALE = 0.005
PF_SCALE = 0.01

_PREC = jax.lax.Precision.HIGHEST


def _rfft_mats(m, nfft):
    nbins = nfft // 2 + 1
    t = np.arange(m, dtype=np.float64)[:, None]
    k = np.arange(nbins, dtype=np.float64)[None, :]
    ang = 2.0 * np.pi * t * k / nfft
    return (np.cos(ang).astype(np.float32),
            (-np.sin(ang)).astype(np.float32))


def _irfft_mats(nfft):
    nbins = nfft // 2 + 1
    k = np.arange(nbins, dtype=np.float64)[:, None]
    t = np.arange(nfft, dtype=np.float64)[None, :]
    ang = 2.0 * np.pi * k * t / nfft
    w = np.where((k == 0) | (k == nfft // 2), 1.0, 2.0) / nfft
    A = (w * np.cos(ang)).astype(np.float32)
    B = (-w * np.sin(ang)).astype(np.float32)
    return A, B


_PB = 576


def _pulse_body(amp_ref, ph_ref, frac_ref, A_ref, B_ref, out_ref):
    amp = amp_ref[...]
    ph = ph_ref[...]
    frac = frac_ref[...]
    k = jax.lax.broadcasted_iota(jnp.int32, amp.shape, 1).astype(jnp.float32)
    ang = ph + k * (-2.0 * jnp.pi / IR_LEN) * frac
    re = amp * jnp.cos(ang)
    im = amp * jnp.sin(ang)
    out_ref[...] = (jnp.dot(re, A_ref[...], precision=_PREC,
                            preferred_element_type=jnp.float32)
                    + jnp.dot(im, B_ref[...], precision=_PREC,
                              preferred_element_type=jnp.float32))


def _synth_pulses(amp, ph, frac, window):
    nfreq = amp.shape[1]
    A, B = _irfft_mats(IR_LEN)
    Aw = jnp.asarray(A) * window[None, :]
    Bw = jnp.asarray(B) * window[None, :]
    nblk = MAX_PULSES // _PB
    return pl.pallas_call(
        _pulse_body,
        grid=(nblk,),
        in_specs=[
            pl.BlockSpec((_PB, nfreq), lambda i: (i, 0)),
            pl.BlockSpec((_PB, nfreq), lambda i: (i, 0)),
            pl.BlockSpec((_PB, 1), lambda i: (i, 0)),
            pl.BlockSpec((nfreq, IR_LEN), lambda i: (0, 0)),
            pl.BlockSpec((nfreq, IR_LEN), lambda i: (0, 0)),
        ],
        out_specs=pl.BlockSpec((_PB, IR_LEN), lambda i: (i, 0)),
        out_shape=jax.ShapeDtypeStruct((MAX_PULSES, IR_LEN), jnp.float32),
        compiler_params=pltpu.CompilerParams(
            dimension_semantics=("parallel",)),
    )(amp, ph, frac[:, None], Aw, Bw)


_NB = 400


def _noise_body(fr_ref, ap_ref, C_ref, S_ref, A_ref, B_ref, out_ref):
    fr = fr_ref[...]
    scale = ap_ref[...]
    re = jnp.dot(fr, C_ref[...], precision=_PREC,
                 preferred_element_type=jnp.float32) * scale
    im = jnp.dot(fr, S_ref[...], precision=_PREC,
                 preferred_element_type=jnp.float32) * scale
    out_ref[...] = (jnp.dot(re, A_ref[...], precision=_PREC,
                            preferred_element_type=jnp.float32)
                    + jnp.dot(im, B_ref[...], precision=_PREC,
                              preferred_element_type=jnp.float32))


def _shape_noise(frames, ap_z):
    rows = frames.shape[0]
    n2 = 2 * HOP
    nbins = HOP + 1
    C, S = _rfft_mats(n2, n2)
    A, B = _irfft_mats(n2)
    n = np.arange(n2, dtype=np.float64)
    hann = (0.5 * (1.0 - np.cos(2.0 * np.pi * n / n2))).astype(np.float32)
    Ah = jnp.asarray(A * hann[None, :])
    Bh = jnp.asarray(B * hann[None, :])
    nblk = rows // _NB
    return pl.pallas_call(
        _noise_body,
        grid=(nblk,),
        in_specs=[
            pl.BlockSpec((_NB, n2), lambda i: (i, 0)),
            pl.BlockSpec((_NB, nbins), lambda i: (i, 0)),
            pl.BlockSpec((n2, nbins), lambda i: (0, 0)),
            pl.BlockSpec((n2, nbins), lambda i: (0, 0)),
            pl.BlockSpec((nbins, n2), lambda i: (0, 0)),
            pl.BlockSpec((nbins, n2), lambda i: (0, 0)),
        ],
        out_specs=pl.BlockSpec((_NB, n2), lambda i: (i, 0)),
        out_shape=jax.ShapeDtypeStruct((rows, n2), jnp.float32),
        compiler_params=pltpu.CompilerParams(
            dimension_semantics=("parallel",)),
    )(frames, ap_z, jnp.asarray(C), jnp.asarray(S), Ah, Bh)


_FB = 400


def _filt_body(sp_ref, sa_ref, pf_ref, Cr_ref, Sr_ref, Cp_ref, Sp_ref,
               A_ref, B_ref, yp_ref, ya_ref):
    pf = pf_ref[...]
    pfre = jnp.dot(pf, Cp_ref[...], precision=_PREC,
                   preferred_element_type=jnp.float32)
    pfim = jnp.dot(pf, Sp_ref[...], precision=_PREC,
                   preferred_element_type=jnp.float32)
    Cr = Cr_ref[...]
    Sr = Sr_ref[...]
    A = A_ref[...]
    B = B_ref[...]
    for s_ref, y_ref in ((sp_ref, yp_ref), (sa_ref, ya_ref)):
        s = s_ref[...]
        xre = jnp.dot(s, Cr, precision=_PREC,
                      preferred_element_type=jnp.float32)
        xim = jnp.dot(s, Sr, precision=_PREC,
                      preferred_element_type=jnp.float32)
        yre = xre * pfre - xim * pfim
        yim = xre * pfim + xim * pfre
        y_ref[...] = (jnp.dot(yre, A, precision=_PREC,
                              preferred_element_type=jnp.float32)
                      + jnp.dot(yim, B, precision=_PREC,
                                preferred_element_type=jnp.float32))


def _post_filter(sigp, siga, pf_rows):
    rows = sigp.shape[0]
    nbins = NFFT_PF // 2 + 1
    Cr, Sr = _rfft_mats(HOP, NFFT_PF)
    Cp, Sp = _rfft_mats(IR_LEN, NFFT_PF)
    A, B = _irfft_mats(NFFT_PF)
    nblk = rows // _FB
    row_spec = lambda w: pl.BlockSpec((_FB, w), lambda i: (i, 0))
    mat_spec = lambda r, c: pl.BlockSpec((r, c), lambda i: (0, 0))
    return pl.pallas_call(
        _filt_body,
        grid=(nblk,),
        in_specs=[
            row_spec(HOP), row_spec(HOP), row_spec(IR_LEN),
            mat_spec(HOP, nbins), mat_spec(HOP, nbins),
            mat_spec(IR_LEN, nbins), mat_spec(IR_LEN, nbins),
            mat_spec(nbins, NFFT_PF), mat_spec(nbins, NFFT_PF),
        ],
        out_specs=[row_spec(NFFT_PF), row_spec(NFFT_PF)],
        out_shape=[jax.ShapeDtypeStruct((rows, NFFT_PF), jnp.float32),
                   jax.ShapeDtypeStruct((rows, NFFT_PF), jnp.float32)],
        compiler_params=pltpu.CompilerParams(
            dimension_semantics=("parallel",)),
    )(sigp, siga, pf_rows, jnp.asarray(Cr), jnp.asarray(Sr),
      jnp.asarray(Cp), jnp.asarray(Sp), jnp.asarray(A), jnp.asarray(B))


def _fold768(y):
    b, L, _ = y.shape
    z = jnp.zeros((b, L + 3, HOP), y.dtype)
    z = z.at[:, 0:L].add(y[:, :, 0:HOP])
    z = z.at[:, 1:L + 1].add(y[:, :, HOP:2 * HOP])
    z = z.at[:, 2:L + 2].add(y[:, :, 2 * HOP:3 * HOP])
    z = z.at[:, 3:L + 3, 0:NFFT_PF - 3 * HOP].add(y[:, :, 3 * HOP:])
    flat = z.reshape(b, (L + 3) * HOP)
    return flat[:, 120:120 + L * HOP]


def kernel(ir, aperiodicity, post_filter, ir_window, pitch):
    b, c, length = ir.shape
    k1, k2 = jax.random.split(jax.random.key(42))
    nfreq = c // 2 + 1

    ir = ir * IR_SCALE
    ir_amp = jnp.exp(ir[:, :nfreq])
    ir_phase = jnp.pad(ir[:, nfreq:], ((0, 0), (1, 1), (0, 0)))
    ir_phase = ir_phase.at[:, 1::2].add(jnp.pi)
    pitch_s = jnp.repeat(pitch.astype(jnp.float32), HOP, axis=1)
    t = pitch_s.shape[1]
    nf = pitch_s / SR
    nf = nf.at[:, 0].set(jax.random.uniform(k1, (b,), dtype=jnp.float32))
    phase = jnp.cumsum(nf, axis=1) % 1.0
    wrap = phase[:, :-1] > phase[:, 1:]
    flat = jnp.nonzero(wrap.ravel(), size=MAX_PULSES, fill_value=0)[0]
    valid = jnp.arange(MAX_PULSES) < jnp.sum(wrap)
    i0 = flat // (t - 1)
    i1 = flat % (t - 1)
    numer = 1.0 - phase[i0, i1]
    denom = numer + phase[i0, i1 + 1]
    frac = numer / jnp.where(denom > 0, denom, 1.0)
    frame = i1 // HOP
    amp = ir_amp[i0, :, frame] * valid[:, None].astype(jnp.float32)
    ph = ir_phase[i0, :, frame]

    pulses = _synth_pulses(amp, ph, frac, ir_window)
    acc = jnp.zeros((b, t + IR_LEN), dtype=jnp.float32)
    acc = acc.at[i0[:, None], i1[:, None] + jnp.arange(IR_LEN)].add(pulses)
    periodic_raw = acc[:, :t]

    ap = aperiodicity * AP_SCALE
    exc = jax.random.uniform(k2, (b, (length + 1) * HOP),
                             dtype=jnp.float32) - 0.5
    e2 = exc.reshape(b, length + 1, HOP)
    frames = jnp.concatenate([e2[:, :-1], e2[:, 1:]], axis=2)
    ap_t = jnp.transpose(ap, (0, 2, 1))
    ap_z = jnp.pad(ap_t, ((0, 0), (0, 0), (1, 0)))
    nz = _shape_noise(frames.reshape(b * length, 2 * HOP),
                      ap_z.reshape(b * length, HOP + 1))
    nz = nz.reshape(b, length, 2 * HOP)
    zo = jnp.zeros((b, length + 1, HOP), jnp.float32)
    zo = zo.at[:, :length].add(nz[:, :, :HOP]).at[:, 1:].add(nz[:, :, HOP:])
    aperiodic_raw = zo.reshape(b, (length + 1) * HOP)[:, :length * HOP]
    noise_exc = exc[:, :length * HOP][:, 120:]

    pf = post_filter * PF_SCALE
    pf = pf.at[:, 0, :].add(1.0)
    pf_rows = jnp.transpose(pf, (0, 2, 1)).reshape(b * length, c)
    yp, ya = _post_filter(periodic_raw.reshape(b * length, HOP),
                          aperiodic_raw.reshape(b * length, HOP),
                          pf_rows)
    periodic = _fold768(yp.reshape(b, length, NFFT_PF))
    aperiodic = _fold768(ya.reshape(b, length, NFFT_PF))
    y = (periodic + aperiodic)[:, None, :]
    return y, periodic, aperiodic, noise_exc


# speedup vs baseline: 6.4467x; 5.6106x over previous
import math

import jax
import jax.numpy as jnp
import numpy as np
from jax.experimental import pallas as pl
from jax.experimental.pallas import tpu as pltpu

HOP = 240
IR_LEN = 512
NFFT_PF = 768
SR = 24000.0
MAX_PULSES = 72000
IR_SCALE = 1.0
AP_SCALE = 0.005
PF_SCALE = 0.01

_PREC = jax.lax.Precision.HIGHEST


def _rfft_mats(m, nfft):
    nbins = nfft // 2 + 1
    t = np.arange(m, dtype=np.float64)[:, None]
    k = np.arange(nbins, dtype=np.float64)[None, :]
    ang = 2.0 * np.pi * t * k / nfft
    return (np.cos(ang).astype(np.float32),
            (-np.sin(ang)).astype(np.float32))


def _irfft_mats(nfft):
    nbins = nfft // 2 + 1
    k = np.arange(nbins, dtype=np.float64)[:, None]
    t = np.arange(nfft, dtype=np.float64)[None, :]
    ang = 2.0 * np.pi * k * t / nfft
    w = np.where((k == 0) | (k == nfft // 2), 1.0, 2.0) / nfft
    A = (w * np.cos(ang)).astype(np.float32)
    B = (-w * np.sin(ang)).astype(np.float32)
    return A, B


_PB = 576


def _pulse_body(amp_ref, ph_ref, frac_ref, A_ref, B_ref, out_ref):
    amp = amp_ref[...]
    ph = ph_ref[...]
    frac = frac_ref[...]
    k = jax.lax.broadcasted_iota(jnp.int32, amp.shape, 1).astype(jnp.float32)
    ang = ph + k * (-2.0 * jnp.pi / IR_LEN) * frac
    re = amp * jnp.cos(ang)
    im = amp * jnp.sin(ang)
    out_ref[...] = (jnp.dot(re, A_ref[...], precision=_PREC,
                            preferred_element_type=jnp.float32)
                    + jnp.dot(im, B_ref[...], precision=_PREC,
                              preferred_element_type=jnp.float32))


def _synth_pulses(amp, ph, frac, window):
    nfreq = amp.shape[1]
    A, B = _irfft_mats(IR_LEN)
    Aw = jnp.asarray(A) * window[None, :]
    Bw = jnp.asarray(B) * window[None, :]
    nblk = MAX_PULSES // _PB
    return pl.pallas_call(
        _pulse_body,
        grid=(nblk,),
        in_specs=[
            pl.BlockSpec((_PB, nfreq), lambda i: (i, 0)),
            pl.BlockSpec((_PB, nfreq), lambda i: (i, 0)),
            pl.BlockSpec((_PB, 1), lambda i: (i, 0)),
            pl.BlockSpec((nfreq, IR_LEN), lambda i: (0, 0)),
            pl.BlockSpec((nfreq, IR_LEN), lambda i: (0, 0)),
        ],
        out_specs=pl.BlockSpec((_PB, IR_LEN), lambda i: (i, 0)),
        out_shape=jax.ShapeDtypeStruct((MAX_PULSES, IR_LEN), jnp.float32),
        compiler_params=pltpu.CompilerParams(
            dimension_semantics=("parallel",)),
    )(amp, ph, frac[:, None], Aw, Bw)


_RB = 400
_SLOTS = 5


def _shift_body(dt_ref, d_ref, C_ref, S_ref, A_ref, B_ref, out_ref):
    nbins = NFFT_PF // 2 + 1
    accre = jnp.zeros((dt_ref.shape[0], nbins), jnp.float32)
    accim = jnp.zeros((dt_ref.shape[0], nbins), jnp.float32)
    k = jax.lax.broadcasted_iota(jnp.int32, (dt_ref.shape[0], nbins),
                                 1).astype(jnp.float32)
    for s in range(_SLOTS):
        x = dt_ref[:, s, :]
        re = jnp.dot(x, C_ref[...], precision=_PREC,
                     preferred_element_type=jnp.float32)
        im = jnp.dot(x, S_ref[...], precision=_PREC,
                     preferred_element_type=jnp.float32)
        ang = k * ((-2.0 * jnp.pi / NFFT_PF) * d_ref[:, s][:, None])
        c = jnp.cos(ang)
        si = jnp.sin(ang)
        accre += re * c - im * si
        accim += re * si + im * c
    out_ref[...] = (jnp.dot(accre, A_ref[...], precision=_PREC,
                            preferred_element_type=jnp.float32)
                    + jnp.dot(accim, B_ref[...], precision=_PREC,
                              preferred_element_type=jnp.float32))


def _shift_ola(dt, d):
    rows = dt.shape[0]
    nbins = NFFT_PF // 2 + 1
    C, S = _rfft_mats(IR_LEN, NFFT_PF)
    A, B = _irfft_mats(NFFT_PF)
    nblk = rows // _RB
    return pl.pallas_call(
        _shift_body,
        grid=(nblk,),
        in_specs=[
            pl.BlockSpec((_RB, _SLOTS, IR_LEN), lambda i: (i, 0, 0)),
            pl.BlockSpec((_RB, _SLOTS), lambda i: (i, 0)),
            pl.BlockSpec((IR_LEN, nbins), lambda i: (0, 0)),
            pl.BlockSpec((IR_LEN, nbins), lambda i: (0, 0)),
            pl.BlockSpec((nbins, NFFT_PF), lambda i: (0, 0)),
            pl.BlockSpec((nbins, NFFT_PF), lambda i: (0, 0)),
        ],
        out_specs=pl.BlockSpec((_RB, NFFT_PF), lambda i: (i, 0)),
        out_shape=jax.ShapeDtypeStruct((rows, NFFT_PF), jnp.float32),
        compiler_params=pltpu.CompilerParams(
            dimension_semantics=("parallel",)),
    )(dt, d, jnp.asarray(C), jnp.asarray(S), jnp.asarray(A), jnp.asarray(B))


_NB = 400


def _noise_body(fr_ref, ap_ref, C_ref, S_ref, A_ref, B_ref, out_ref):
    fr = fr_ref[...]
    scale = ap_ref[...]
    re = jnp.dot(fr, C_ref[...], precision=_PREC,
                 preferred_element_type=jnp.float32) * scale
    im = jnp.dot(fr, S_ref[...], precision=_PREC,
                 preferred_element_type=jnp.float32) * scale
    out_ref[...] = (jnp.dot(re, A_ref[...], precision=_PREC,
                            preferred_element_type=jnp.float32)
                    + jnp.dot(im, B_ref[...], precision=_PREC,
                              preferred_element_type=jnp.float32))


def _shape_noise(frames, ap_z):
    rows = frames.shape[0]
    n2 = 2 * HOP
    nbins = HOP + 1
    C, S = _rfft_mats(n2, n2)
    A, B = _irfft_mats(n2)
    n = np.arange(n2, dtype=np.float64)
    hann = (0.5 * (1.0 - np.cos(2.0 * np.pi * n / n2))).astype(np.float32)
    Ah = jnp.asarray(A * hann[None, :])
    Bh = jnp.asarray(B * hann[None, :])
    nblk = rows // _NB
    return pl.pallas_call(
        _noise_body,
        grid=(nblk,),
        in_specs=[
            pl.BlockSpec((_NB, n2), lambda i: (i, 0)),
            pl.BlockSpec((_NB, nbins), lambda i: (i, 0)),
            pl.BlockSpec((n2, nbins), lambda i: (0, 0)),
            pl.BlockSpec((n2, nbins), lambda i: (0, 0)),
            pl.BlockSpec((nbins, n2), lambda i: (0, 0)),
            pl.BlockSpec((nbins, n2), lambda i: (0, 0)),
        ],
        out_specs=pl.BlockSpec((_NB, n2), lambda i: (i, 0)),
        out_shape=jax.ShapeDtypeStruct((rows, n2), jnp.float32),
        compiler_params=pltpu.CompilerParams(
            dimension_semantics=("parallel",)),
    )(frames, ap_z, jnp.asarray(C), jnp.asarray(S), Ah, Bh)


_FB = 400


def _filt_body(sp_ref, sa_ref, pf_ref, Cr_ref, Sr_ref, Cp_ref, Sp_ref,
               A_ref, B_ref, yp_ref, ya_ref):
    pf = pf_ref[...]
    pfre = jnp.dot(pf, Cp_ref[...], precision=_PREC,
                   preferred_element_type=jnp.float32)
    pfim = jnp.dot(pf, Sp_ref[...], precision=_PREC,
                   preferred_element_type=jnp.float32)
    Cr = Cr_ref[...]
    Sr = Sr_ref[...]
    A = A_ref[...]
    B = B_ref[...]
    for s_ref, y_ref in ((sp_ref, yp_ref), (sa_ref, ya_ref)):
        s = s_ref[...]
        xre = jnp.dot(s, Cr, precision=_PREC,
                      preferred_element_type=jnp.float32)
        xim = jnp.dot(s, Sr, precision=_PREC,
                      preferred_element_type=jnp.float32)
        yre = xre * pfre - xim * pfim
        yim = xre * pfim + xim * pfre
        y_ref[...] = (jnp.dot(yre, A, precision=_PREC,
                              preferred_element_type=jnp.float32)
                      + jnp.dot(yim, B, precision=_PREC,
                                preferred_element_type=jnp.float32))


def _post_filter(sigp, siga, pf_rows):
    rows = sigp.shape[0]
    nbins = NFFT_PF // 2 + 1
    Cr, Sr = _rfft_mats(HOP, NFFT_PF)
    Cp, Sp = _rfft_mats(IR_LEN, NFFT_PF)
    A, B = _irfft_mats(NFFT_PF)
    nblk = rows // _FB
    row_spec = lambda w: pl.BlockSpec((_FB, w), lambda i: (i, 0))
    mat_spec = lambda r, c: pl.BlockSpec((r, c), lambda i: (0, 0))
    return pl.pallas_call(
        _filt_body,
        grid=(nblk,),
        in_specs=[
            row_spec(HOP), row_spec(HOP), row_spec(IR_LEN),
            mat_spec(HOP, nbins), mat_spec(HOP, nbins),
            mat_spec(IR_LEN, nbins), mat_spec(IR_LEN, nbins),
            mat_spec(nbins, NFFT_PF), mat_spec(nbins, NFFT_PF),
        ],
        out_specs=[row_spec(NFFT_PF), row_spec(NFFT_PF)],
        out_shape=[jax.ShapeDtypeStruct((rows, NFFT_PF), jnp.float32),
                   jax.ShapeDtypeStruct((rows, NFFT_PF), jnp.float32)],
        compiler_params=pltpu.CompilerParams(
            dimension_semantics=("parallel",)),
    )(sigp, siga, pf_rows, jnp.asarray(Cr), jnp.asarray(Sr),
      jnp.asarray(Cp), jnp.asarray(Sp), jnp.asarray(A), jnp.asarray(B))


def _fold768(y):
    b, L, _ = y.shape
    z = jnp.zeros((b, L + 3, HOP), y.dtype)
    z = z.at[:, 0:L].add(y[:, :, 0:HOP])
    z = z.at[:, 1:L + 1].add(y[:, :, HOP:2 * HOP])
    z = z.at[:, 2:L + 2].add(y[:, :, 2 * HOP:3 * HOP])
    z = z.at[:, 3:L + 3, 0:NFFT_PF - 3 * HOP].add(y[:, :, 3 * HOP:])
    flat = z.reshape(b, (L + 3) * HOP)
    return flat[:, 120:120 + L * HOP]


def kernel(ir, aperiodicity, post_filter, ir_window, pitch):
    b, c, length = ir.shape
    k1, k2 = jax.random.split(jax.random.key(42))
    nfreq = c // 2 + 1

    ir = ir * IR_SCALE
    ir_amp = jnp.exp(ir[:, :nfreq])
    ir_phase = jnp.pad(ir[:, nfreq:], ((0, 0), (1, 1), (0, 0)))
    ir_phase = ir_phase.at[:, 1::2].add(jnp.pi)
    pitch_s = jnp.repeat(pitch.astype(jnp.float32), HOP, axis=1)
    t = pitch_s.shape[1]
    nf = pitch_s / SR
    nf = nf.at[:, 0].set(jax.random.uniform(k1, (b,), dtype=jnp.float32))
    phase = jnp.cumsum(nf, axis=1) % 1.0
    wrap = phase[:, :-1] > phase[:, 1:]
    flat = jnp.nonzero(wrap.ravel(), size=MAX_PULSES, fill_value=0)[0]
    valid = jnp.arange(MAX_PULSES) < jnp.sum(wrap)
    i0 = flat // (t - 1)
    i1 = flat % (t - 1)
    numer = 1.0 - phase[i0, i1]
    denom = numer + phase[i0, i1 + 1]
    frac = numer / jnp.where(denom > 0, denom, 1.0)
    frame = i1 // HOP
    amp = ir_amp[i0, :, frame] * valid[:, None].astype(jnp.float32)
    ph = ir_phase[i0, :, frame]

    pulses = _synth_pulses(amp, ph, frac, ir_window)

    nrow = b * length
    bkey = jnp.where(valid, i0 * length + frame, nrow)
    starts = jnp.searchsorted(bkey, jnp.arange(nrow), side="left")
    q = starts[:, None] + jnp.arange(_SLOTS)[None, :]
    qc = jnp.minimum(q, MAX_PULSES - 1)
    ok = (q < MAX_PULSES) & (bkey[qc] == jnp.arange(nrow)[:, None])
    dt = pulses[qc] * ok[..., None].astype(jnp.float32)
    dd = jnp.where(ok, (i1[qc] % HOP).astype(jnp.float32), 0.0)
    seg = _shift_ola(dt, dd).reshape(b, length, NFFT_PF)
    zp = jnp.zeros((b, length + 3, HOP), jnp.float32)
    zp = zp.at[:, 0:length].add(seg[:, :, 0:HOP])
    zp = zp.at[:, 1:length + 1].add(seg[:, :, HOP:2 * HOP])
    zp = zp.at[:, 2:length + 2].add(seg[:, :, 2 * HOP:3 * HOP])
    zp = zp.at[:, 3:length + 3, 0:NFFT_PF - 3 * HOP].add(seg[:, :, 3 * HOP:])
    periodic_raw = zp.reshape(b, (length + 3) * HOP)[:, :t]

    ap = aperiodicity * AP_SCALE
    exc = jax.random.uniform(k2, (b, (length + 1) * HOP),
                             dtype=jnp.float32) - 0.5
    e2 = exc.reshape(b, length + 1, HOP)
    frames = jnp.concatenate([e2[:, :-1], e2[:, 1:]], axis=2)
    ap_t = jnp.transpose(ap, (0, 2, 1))
    ap_z = jnp.pad(ap_t, ((0, 0), (0, 0), (1, 0)))
    nz = _shape_noise(frames.reshape(b * length, 2 * HOP),
                      ap_z.reshape(b * length, HOP + 1))
    nz = nz.reshape(b, length, 2 * HOP)
    zo = jnp.zeros((b, length + 1, HOP), jnp.float32)
    zo = zo.at[:, :length].add(nz[:, :, :HOP]).at[:, 1:].add(nz[:, :, HOP:])
    aperiodic_raw = zo.reshape(b, (length + 1) * HOP)[:, :length * HOP]
    noise_exc = exc[:, :length * HOP][:, 120:]

    pf = post_filter * PF_SCALE
    pf = pf.at[:, 0, :].add(1.0)
    pf_rows = jnp.transpose(pf, (0, 2, 1)).reshape(b * length, c)
    yp, ya = _post_filter(periodic_raw.reshape(b * length, HOP),
                          aperiodic_raw.reshape(b * length, HOP),
                          pf_rows)
    periodic = _fold768(yp.reshape(b, length, NFFT_PF))
    aperiodic = _fold768(ya.reshape(b, length, NFFT_PF))
    y = (periodic + aperiodic)[:, None, :]
    return y, periodic, aperiodic, noise_exc
